# Initial kernel scaffold; baseline (speedup 1.0000x reference)
import jax
import jax.numpy as jnp
from jax.experimental import pallas as pl


def kernel(x, Wg, W1, b1, W2, b2):
    raise NotImplementedError("write your pallas kernel here")


# speedup vs baseline: 1.2357x; 1.2357x over previous
import functools

import jax
import jax.numpy as jnp
from jax import lax
from jax.experimental import pallas as pl
from jax.experimental.pallas import tpu as pltpu
from jax.experimental.pallas import tpu_sc as plsc

CAP_FACTOR_NUM = 5
CAP_FACTOR_DEN = 4

NC = 2
NS = 16
NW = NC * NS

PB = 1024
HBLK = 512
BT = 512
C = 64


def _gate_body(x_ref, wg_ref, i1_ref, i2_ref, v1_ref, v2_ref, gs_ref):
    xb = x_ref[...].astype(jnp.bfloat16)
    wb = wg_ref[...].astype(jnp.bfloat16)
    logits = jnp.dot(xb, wb, preferred_element_type=jnp.float32)
    m = jnp.max(logits, axis=1, keepdims=True)
    p = jnp.exp(logits - m)
    g = p / jnp.sum(p, axis=1, keepdims=True)
    e = g.shape[1]
    cols = lax.broadcasted_iota(jnp.int32, g.shape, 1)
    big = jnp.int32(e)
    v1 = jnp.max(g, axis=1, keepdims=True)
    i1 = jnp.min(jnp.where(g == v1, cols, big), axis=1, keepdims=True)
    g2 = jnp.where(cols == i1, -jnp.inf, g)
    v2 = jnp.max(g2, axis=1, keepdims=True)
    i2 = jnp.min(jnp.where(g2 == v2, cols, big), axis=1, keepdims=True)
    i1_ref[...] = i1
    i2_ref[...] = i2
    v1_ref[...] = v1
    v2_ref[...] = v2
    gs_ref[...] = jnp.sum(g, axis=0, keepdims=True)[None]


def _positions_body(i1_ref, i2_ref, v1_ref, v2_ref, gs_ref,
                    dstd_ref, dstc_ref, w_ref, cnt_ref, laux_ref,
                    tri_ref, off_ref, cnt1_ref, *, cap, e_num, t_num, nb):
    k = pl.program_id(0)
    b = pl.program_id(1)

    @pl.when(jnp.logical_and(k == 0, b == 0))
    def _init():
        rr = lax.broadcasted_iota(jnp.int32, (PB, PB), 0)
        cc = lax.broadcasted_iota(jnp.int32, (PB, PB), 1)
        tri_ref[...] = (rr >= cc).astype(jnp.bfloat16)
        off_ref[...] = jnp.zeros_like(off_ref)
        cnt1_ref[...] = jnp.zeros_like(cnt1_ref)

    ik = jnp.where(k == 0, i1_ref[...], i2_ref[...])
    vk = jnp.where(k == 0, v1_ref[...], v2_ref[...])
    cols = lax.broadcasted_iota(jnp.int32, (PB, e_num), 1)
    mh = (cols == ik)
    m_bf = mh.astype(jnp.bfloat16)
    m_f = mh.astype(jnp.float32)
    cs = jnp.dot(tri_ref[...], m_bf, preferred_element_type=jnp.float32)
    off = off_ref[...]
    pos_in = cs - 1.0 + off
    posf = jnp.sum(m_f * pos_in, axis=1, keepdims=True)
    posi = posf.astype(jnp.int32)
    valid = posi < cap
    dummy = jnp.int32(e_num * cap)
    dstd_ref[...] = jnp.where(valid, ik * cap + posi, dummy)[None, None]
    dstc_ref[...] = (ik * cap + jnp.minimum(posi, cap - 1))[None, None]
    w_ref[...] = (vk * valid.astype(jnp.float32))[None, None]
    off = off + cs[PB - 1:PB, :]
    off_ref[...] = off
    cnt_ref[...] = off[None]

    @pl.when(jnp.logical_and(k == 0, b == nb - 1))
    def _snap():
        cnt1_ref[...] = off

    gs_tot = jnp.sum(gs_ref[...][:, 0, :], axis=0, keepdims=True)
    laux = jnp.sum(gs_tot * cnt1_ref[...], axis=1, keepdims=True)
    laux_ref[...] = laux * (e_num / (float(t_num) * float(t_num)))


def _ffn_body(cnt_ref, buf_ref, w1_ref, b1_ref, w2_ref, b2_ref, out_ref,
              acc_ref, xb_ref, *, cap, d, hb_num):
    e = pl.program_id(0)
    hb = pl.program_id(1)

    @pl.when(hb == 0)
    def _load():
        n = jnp.minimum(cnt_ref[e], cap)
        rows = lax.broadcasted_iota(jnp.int32, (cap, d), 0)
        xb_ref[...] = jnp.where(rows < n, buf_ref[...], 0.0).astype(jnp.bfloat16)

    h = jnp.dot(xb_ref[...], w1_ref[0], preferred_element_type=jnp.float32)
    h = jax.nn.gelu(h + b1_ref[0, 0], approximate=True)
    part = jnp.dot(h.astype(jnp.bfloat16), w2_ref[0],
                   preferred_element_type=jnp.float32)

    @pl.when(hb == 0)
    def _first():
        acc_ref[...] = part

    @pl.when(hb > 0)
    def _rest():
        acc_ref[...] += part

    @pl.when(hb == hb_num - 1)
    def _store():
        out_ref[...] = acc_ref[...] + b2_ref[0]


def _combine_body(w_ref, r_ref, y_ref):
    y_ref[...] = (w_ref[0] * r_ref[0] + w_ref[1] * r_ref[1])


def _dispatch_sc(nchunk, t_num, d, buf_rows):
    mesh = plsc.VectorSubcoreMesh(core_axis_name="c", subcore_axis_name="s",
                                  num_cores=NC, num_subcores=NS)
    tpw = t_num // NW

    @functools.partial(
        pl.kernel,
        out_type=jax.ShapeDtypeStruct((buf_rows, d), jnp.float32),
        mesh=mesh,
        scratch_types=[
            pltpu.VMEM((C, d), jnp.float32),
            pltpu.VMEM((2 * nchunk, C), jnp.int32),
            pltpu.SemaphoreType.DMA,
        ],
    )
    def dispatch(x_hbm, idx_hbm, buf_hbm, xbuf, idx_v, sem):
        wid = lax.axis_index("s") * NC + lax.axis_index("c")
        pltpu.sync_copy(idx_hbm.at[wid], idx_v)
        for c in range(nchunk):
            base = wid * tpw + c * C
            pltpu.sync_copy(x_hbm.at[pl.ds(base, C)], xbuf)
            a0 = pltpu.async_copy(xbuf, buf_hbm.at[idx_v.at[2 * c]], sem)
            a1 = pltpu.async_copy(xbuf, buf_hbm.at[idx_v.at[2 * c + 1]], sem)
            a0.wait()
            a1.wait()

    return dispatch


def _gather_sc(nchunk, t_num, d, out_rows):
    mesh = plsc.VectorSubcoreMesh(core_axis_name="c", subcore_axis_name="s",
                                  num_cores=NC, num_subcores=NS)
    tpw = t_num // NW

    @functools.partial(
        pl.kernel,
        out_type=jax.ShapeDtypeStruct((2 * t_num, d), jnp.float32),
        mesh=mesh,
        scratch_types=[
            pltpu.VMEM((C, d), jnp.float32),
            pltpu.VMEM((2 * nchunk, C), jnp.int32),
            pltpu.SemaphoreType.DMA,
        ],
    )
    def gather(ob_hbm, idx_hbm, r_hbm, rowbuf, idx_v, sem):
        wid = lax.axis_index("s") * NC + lax.axis_index("c")
        pltpu.sync_copy(idx_hbm.at[wid], idx_v)
        for c in range(nchunk):
            for k in range(2):
                cp = pltpu.async_copy(ob_hbm.at[idx_v.at[2 * c + k]],
                                      rowbuf, sem)
                cp.wait()
                dst = k * t_num + wid * tpw + c * C
                pltpu.sync_copy(rowbuf, r_hbm.at[pl.ds(dst, C)])

    return gather


def _worker_layout(a, t_num, nchunk):
    return (a.reshape(2, NW, nchunk, C)
             .transpose(1, 2, 0, 3)
             .reshape(NW, nchunk * 2, C))


def kernel(x, Wg, W1, b1, W2, b2):
    t_num, d = x.shape
    e_num = Wg.shape[1]
    h_num = W1.shape[2]
    cap = 2 * ((CAP_FACTOR_NUM * ((t_num + e_num - 1) // e_num))
               // CAP_FACTOR_DEN)
    nb = t_num // PB
    hb_num = h_num // HBLK
    nchunk = t_num // (NW * C)
    buf_rows = e_num * cap + 8

    f32 = jnp.float32
    i32 = jnp.int32

    i1, i2, v1, v2, gs = pl.pallas_call(
        _gate_body,
        grid=(nb,),
        in_specs=[
            pl.BlockSpec((PB, d), lambda b: (b, 0)),
            pl.BlockSpec((d, e_num), lambda b: (0, 0)),
        ],
        out_specs=[
            pl.BlockSpec((PB, 1), lambda b: (b, 0)),
            pl.BlockSpec((PB, 1), lambda b: (b, 0)),
            pl.BlockSpec((PB, 1), lambda b: (b, 0)),
            pl.BlockSpec((PB, 1), lambda b: (b, 0)),
            pl.BlockSpec((1, 1, e_num), lambda b: (b, 0, 0)),
        ],
        out_shape=[
            jax.ShapeDtypeStruct((t_num, 1), i32),
            jax.ShapeDtypeStruct((t_num, 1), i32),
            jax.ShapeDtypeStruct((t_num, 1), f32),
            jax.ShapeDtypeStruct((t_num, 1), f32),
            jax.ShapeDtypeStruct((nb, 1, e_num), f32),
        ],
    )(x, Wg)

    pos_body = functools.partial(_positions_body, cap=cap, e_num=e_num,
                                 t_num=t_num, nb=nb)
    dstd, dstc, w, cnt, laux = pl.pallas_call(
        pos_body,
        grid=(2, nb),
        in_specs=[
            pl.BlockSpec((PB, 1), lambda k, b: (b, 0)),
            pl.BlockSpec((PB, 1), lambda k, b: (b, 0)),
            pl.BlockSpec((PB, 1), lambda k, b: (b, 0)),
            pl.BlockSpec((PB, 1), lambda k, b: (b, 0)),
            pl.BlockSpec((nb, 1, e_num), lambda k, b: (0, 0, 0)),
        ],
        out_specs=[
            pl.BlockSpec((1, 1, PB, 1), lambda k, b: (k, b, 0, 0)),
            pl.BlockSpec((1, 1, PB, 1), lambda k, b: (k, b, 0, 0)),
            pl.BlockSpec((1, 1, PB, 1), lambda k, b: (k, b, 0, 0)),
            pl.BlockSpec((1, 1, e_num), lambda k, b: (k, 0, 0)),
            pl.BlockSpec((1, 1), lambda k, b: (0, 0)),
        ],
        out_shape=[
            jax.ShapeDtypeStruct((2, nb, PB, 1), i32),
            jax.ShapeDtypeStruct((2, nb, PB, 1), i32),
            jax.ShapeDtypeStruct((2, nb, PB, 1), f32),
            jax.ShapeDtypeStruct((2, 1, e_num), f32),
            jax.ShapeDtypeStruct((1, 1), f32),
        ],
        scratch_shapes=[
            pltpu.VMEM((PB, PB), jnp.bfloat16),
            pltpu.VMEM((1, e_num), f32),
            pltpu.VMEM((1, e_num), f32),
        ],
    )(i1, i2, v1, v2, gs)

    cnt_tot = cnt[1, 0].astype(i32)

    dstd_l = _worker_layout(dstd.reshape(2, t_num), t_num, nchunk)
    buf = _dispatch_sc(nchunk, t_num, d, buf_rows)(x, dstd_l)

    ffn_body = functools.partial(_ffn_body, cap=cap, d=d, hb_num=hb_num)
    grid_spec = pltpu.PrefetchScalarGridSpec(
        num_scalar_prefetch=1,
        grid=(e_num, hb_num),
        in_specs=[
            pl.BlockSpec((cap, d), lambda e, hb, cnt: (e, 0)),
            pl.BlockSpec((1, d, HBLK), lambda e, hb, cnt: (e, 0, hb)),
            pl.BlockSpec((1, 1, 1, HBLK), lambda e, hb, cnt: (e, hb, 0, 0)),
            pl.BlockSpec((1, HBLK, d), lambda e, hb, cnt: (e, hb, 0)),
            pl.BlockSpec((1, 1, d), lambda e, hb, cnt: (e, 0, 0)),
        ],
        out_specs=pl.BlockSpec((cap, d), lambda e, hb, cnt: (e, 0)),
        scratch_shapes=[
            pltpu.VMEM((cap, d), f32),
            pltpu.VMEM((cap, d), jnp.bfloat16),
        ],
    )
    out_buf = pl.pallas_call(
        ffn_body,
        grid_spec=grid_spec,
        out_shape=jax.ShapeDtypeStruct((e_num * cap, d), f32),
    )(cnt_tot, buf, W1.astype(jnp.bfloat16),
      b1.reshape(e_num, hb_num, 1, HBLK),
      W2.astype(jnp.bfloat16), b2.reshape(e_num, 1, d))

    dstc_l = _worker_layout(dstc.reshape(2, t_num), t_num, nchunk)
    r = _gather_sc(nchunk, t_num, d, e_num * cap)(out_buf, dstc_l)
    r = r.reshape(2, t_num, d)
    wt = w.reshape(2, t_num, 1)

    y = pl.pallas_call(
        _combine_body,
        grid=(t_num // BT,),
        in_specs=[
            pl.BlockSpec((2, BT, 1), lambda i: (0, i, 0)),
            pl.BlockSpec((2, BT, d), lambda i: (0, i, 0)),
        ],
        out_specs=pl.BlockSpec((BT, d), lambda i: (i, 0)),
        out_shape=jax.ShapeDtypeStruct((t_num, d), f32),
    )(wt, r)

    return y, laux[0, 0]


# speedup vs baseline: 1.7825x; 1.4426x over previous
import functools

import jax
import jax.numpy as jnp
from jax import lax
from jax.experimental import pallas as pl
from jax.experimental.pallas import tpu as pltpu
from jax.experimental.pallas import tpu_sc as plsc

CAP_FACTOR_NUM = 5
CAP_FACTOR_DEN = 4

NC = 2
NS = 16
NW = NC * NS

PB = 1024
HBLK = 1024
BT = 512
C = 32


def _gate_body(x_ref, wg_ref, i1_ref, i2_ref, v1_ref, v2_ref, gs_ref):
    xb = x_ref[...].astype(jnp.bfloat16)
    wb = wg_ref[...].astype(jnp.bfloat16)
    logits = jnp.dot(xb, wb, preferred_element_type=jnp.float32)
    m = jnp.max(logits, axis=1, keepdims=True)
    p = jnp.exp(logits - m)
    g = p / jnp.sum(p, axis=1, keepdims=True)
    e = g.shape[1]
    cols = lax.broadcasted_iota(jnp.int32, g.shape, 1)
    big = jnp.int32(e)
    v1 = jnp.max(g, axis=1, keepdims=True)
    i1 = jnp.min(jnp.where(g == v1, cols, big), axis=1, keepdims=True)
    g2 = jnp.where(cols == i1, -jnp.inf, g)
    v2 = jnp.max(g2, axis=1, keepdims=True)
    i2 = jnp.min(jnp.where(g2 == v2, cols, big), axis=1, keepdims=True)
    i1_ref[...] = i1
    i2_ref[...] = i2
    v1_ref[...] = v1
    v2_ref[...] = v2
    gs_ref[...] = jnp.sum(g, axis=0, keepdims=True)[None]


def _positions_body(i1_ref, i2_ref, v1_ref, v2_ref, gs_ref,
                    dstd_ref, dstc_ref, w_ref, cnt_ref, laux_ref,
                    tri_ref, off_ref, cnt1_ref, *, cap, e_num, t_num, nb):
    k = pl.program_id(0)
    b = pl.program_id(1)

    @pl.when(jnp.logical_and(k == 0, b == 0))
    def _init():
        rr = lax.broadcasted_iota(jnp.int32, (PB, PB), 0)
        cc = lax.broadcasted_iota(jnp.int32, (PB, PB), 1)
        tri_ref[...] = (rr >= cc).astype(jnp.bfloat16)
        off_ref[...] = jnp.zeros_like(off_ref)
        cnt1_ref[...] = jnp.zeros_like(cnt1_ref)

    ik = jnp.where(k == 0, i1_ref[...], i2_ref[...])
    vk = jnp.where(k == 0, v1_ref[...], v2_ref[...])
    cols = lax.broadcasted_iota(jnp.int32, (PB, e_num), 1)
    mh = (cols == ik)
    m_bf = mh.astype(jnp.bfloat16)
    m_f = mh.astype(jnp.float32)
    cs = jnp.dot(tri_ref[...], m_bf, preferred_element_type=jnp.float32)
    off = off_ref[...]
    pos_in = cs - 1.0 + off
    posf = jnp.sum(m_f * pos_in, axis=1, keepdims=True)
    posi = posf.astype(jnp.int32)
    valid = posi < cap
    dummy = jnp.int32(e_num * cap)
    dstd_ref[...] = jnp.where(valid, ik * cap + posi, dummy)[None, None]
    dstc_ref[...] = (ik * cap + jnp.minimum(posi, cap - 1))[None, None]
    w_ref[...] = (vk * valid.astype(jnp.float32))[None, None]
    off = off + cs[PB - 1:PB, :]
    off_ref[...] = off
    cnt_ref[...] = off[None]

    @pl.when(jnp.logical_and(k == 0, b == nb - 1))
    def _snap():
        cnt1_ref[...] = off

    gs_tot = jnp.sum(gs_ref[...][:, 0, :], axis=0, keepdims=True)
    laux = jnp.sum(gs_tot * cnt1_ref[...], axis=1, keepdims=True)
    laux_ref[...] = laux * (e_num / (float(t_num) * float(t_num)))


def _ffn_body(cnt_ref, buf_ref, w1_ref, b1_ref, w2_ref, b2_ref, out_ref,
              xb_ref, *, cap, d, hb_num):
    e = pl.program_id(0)
    hb = pl.program_id(1)

    @pl.when(hb == 0)
    def _load():
        n = jnp.minimum(cnt_ref[e], cap)
        rows = lax.broadcasted_iota(jnp.int32, (cap, d), 0)
        xb_ref[...] = jnp.where(rows < n, buf_ref[...],
                                0.0).astype(jnp.bfloat16)

    h = jnp.dot(xb_ref[...], w1_ref[0].astype(jnp.bfloat16),
                preferred_element_type=jnp.float32)
    h = jax.nn.gelu(h + b1_ref[0, 0], approximate=True)
    part = jnp.dot(h.astype(jnp.bfloat16), w2_ref[0].astype(jnp.bfloat16),
                   preferred_element_type=jnp.float32)

    @pl.when(hb == 0)
    def _first():
        out_ref[...] = part + b2_ref[0]

    @pl.when(hb > 0)
    def _rest():
        out_ref[...] += part


def _combine_body(w_ref, r_ref, y_ref):
    y_ref[...] = (w_ref[0] * r_ref[0] + w_ref[1] * r_ref[1])


def _dispatch_sc(nchunk, t_num, d, buf_rows):
    mesh = plsc.VectorSubcoreMesh(core_axis_name="c", subcore_axis_name="s",
                                  num_cores=NC, num_subcores=NS)
    tpw = t_num // NW

    @functools.partial(
        pl.kernel,
        out_type=jax.ShapeDtypeStruct((buf_rows, d), jnp.float32),
        mesh=mesh,
        scratch_types=[
            pltpu.VMEM((2, C, d), jnp.float32),
            pltpu.VMEM((2 * nchunk, C), jnp.int32),
            pltpu.SemaphoreType.DMA,
            pltpu.SemaphoreType.DMA,
            pltpu.SemaphoreType.DMA,
        ],
    )
    def dispatch(x_hbm, idx_hbm, buf_hbm, xbuf, idx_v, isem, lsem, ssem):
        wid = lax.axis_index("s") * NC + lax.axis_index("c")
        icps = []
        for c in range(nchunk):
            for k in range(2):
                icps.append(pltpu.async_copy(
                    idx_hbm.at[k, pl.ds(wid * tpw + c * C, C)],
                    idx_v.at[2 * c + k], isem))
        for cp in icps:
            cp.wait()
        ld = pltpu.async_copy(x_hbm.at[pl.ds(wid * tpw, C)],
                              xbuf.at[0], lsem)
        for c in range(nchunk):
            ld.wait()
            a0 = pltpu.async_copy(xbuf.at[c % 2],
                                  buf_hbm.at[idx_v.at[2 * c]], ssem)
            a1 = pltpu.async_copy(xbuf.at[c % 2],
                                  buf_hbm.at[idx_v.at[2 * c + 1]], ssem)
            if c + 1 < nchunk:
                ld = pltpu.async_copy(
                    x_hbm.at[pl.ds(wid * tpw + (c + 1) * C, C)],
                    xbuf.at[(c + 1) % 2], lsem)
            a0.wait()
            a1.wait()

    return dispatch


def _gather_sc(nchunk, t_num, d, out_rows):
    mesh = plsc.VectorSubcoreMesh(core_axis_name="c", subcore_axis_name="s",
                                  num_cores=NC, num_subcores=NS)
    tpw = t_num // NW

    @functools.partial(
        pl.kernel,
        out_type=jax.ShapeDtypeStruct((2 * t_num, d), jnp.float32),
        mesh=mesh,
        scratch_types=[
            pltpu.VMEM((2, C, d), jnp.float32),
            pltpu.VMEM((2 * nchunk, C), jnp.int32),
            pltpu.SemaphoreType.DMA,
            pltpu.SemaphoreType.DMA,
        ],
    )
    def gather(ob_hbm, idx_hbm, r_hbm, rowbuf, idx_v, isem, gsem):
        wid = lax.axis_index("s") * NC + lax.axis_index("c")
        icps = []
        for c in range(nchunk):
            for k in range(2):
                icps.append(pltpu.async_copy(
                    idx_hbm.at[k, pl.ds(wid * tpw + c * C, C)],
                    idx_v.at[2 * c + k], isem))
        for cp in icps:
            cp.wait()
        g = pltpu.async_copy(ob_hbm.at[idx_v.at[0]], rowbuf.at[0], gsem)
        for j in range(2 * nchunk):
            c, k = j // 2, j % 2
            g.wait()
            if j + 1 < 2 * nchunk:
                g = pltpu.async_copy(ob_hbm.at[idx_v.at[j + 1]],
                                     rowbuf.at[(j + 1) % 2], gsem)
            dst = k * t_num + wid * tpw + c * C
            pltpu.sync_copy(rowbuf.at[j % 2], r_hbm.at[pl.ds(dst, C)])

    return gather


def kernel(x, Wg, W1, b1, W2, b2):
    t_num, d = x.shape
    e_num = Wg.shape[1]
    h_num = W1.shape[2]
    cap = 2 * ((CAP_FACTOR_NUM * ((t_num + e_num - 1) // e_num))
               // CAP_FACTOR_DEN)
    nb = t_num // PB
    hb_num = h_num // HBLK
    nchunk = t_num // (NW * C)
    buf_rows = e_num * cap + 8

    f32 = jnp.float32
    i32 = jnp.int32

    i1, i2, v1, v2, gs = pl.pallas_call(
        _gate_body,
        grid=(nb,),
        in_specs=[
            pl.BlockSpec((PB, d), lambda b: (b, 0)),
            pl.BlockSpec((d, e_num), lambda b: (0, 0)),
        ],
        out_specs=[
            pl.BlockSpec((PB, 1), lambda b: (b, 0)),
            pl.BlockSpec((PB, 1), lambda b: (b, 0)),
            pl.BlockSpec((PB, 1), lambda b: (b, 0)),
            pl.BlockSpec((PB, 1), lambda b: (b, 0)),
            pl.BlockSpec((1, 1, e_num), lambda b: (b, 0, 0)),
        ],
        out_shape=[
            jax.ShapeDtypeStruct((t_num, 1), i32),
            jax.ShapeDtypeStruct((t_num, 1), i32),
            jax.ShapeDtypeStruct((t_num, 1), f32),
            jax.ShapeDtypeStruct((t_num, 1), f32),
            jax.ShapeDtypeStruct((nb, 1, e_num), f32),
        ],
    )(x, Wg)

    pos_body = functools.partial(_positions_body, cap=cap, e_num=e_num,
                                 t_num=t_num, nb=nb)
    dstd, dstc, w, cnt, laux = pl.pallas_call(
        pos_body,
        grid=(2, nb),
        in_specs=[
            pl.BlockSpec((PB, 1), lambda k, b: (b, 0)),
            pl.BlockSpec((PB, 1), lambda k, b: (b, 0)),
            pl.BlockSpec((PB, 1), lambda k, b: (b, 0)),
            pl.BlockSpec((PB, 1), lambda k, b: (b, 0)),
            pl.BlockSpec((nb, 1, e_num), lambda k, b: (0, 0, 0)),
        ],
        out_specs=[
            pl.BlockSpec((1, 1, PB, 1), lambda k, b: (k, b, 0, 0)),
            pl.BlockSpec((1, 1, PB, 1), lambda k, b: (k, b, 0, 0)),
            pl.BlockSpec((1, 1, PB, 1), lambda k, b: (k, b, 0, 0)),
            pl.BlockSpec((1, 1, e_num), lambda k, b: (k, 0, 0)),
            pl.BlockSpec((1, 1), lambda k, b: (0, 0)),
        ],
        out_shape=[
            jax.ShapeDtypeStruct((2, nb, PB, 1), i32),
            jax.ShapeDtypeStruct((2, nb, PB, 1), i32),
            jax.ShapeDtypeStruct((2, nb, PB, 1), f32),
            jax.ShapeDtypeStruct((2, 1, e_num), f32),
            jax.ShapeDtypeStruct((1, 1), f32),
        ],
        scratch_shapes=[
            pltpu.VMEM((PB, PB), jnp.bfloat16),
            pltpu.VMEM((1, e_num), f32),
            pltpu.VMEM((1, e_num), f32),
        ],
    )(i1, i2, v1, v2, gs)

    cnt_tot = cnt[1, 0].astype(i32)

    buf = _dispatch_sc(nchunk, t_num, d, buf_rows)(x, dstd.reshape(2, t_num))

    ffn_body = functools.partial(_ffn_body, cap=cap, d=d, hb_num=hb_num)
    grid_spec = pltpu.PrefetchScalarGridSpec(
        num_scalar_prefetch=1,
        grid=(e_num, hb_num),
        in_specs=[
            pl.BlockSpec((cap, d), lambda e, hb, cnt: (e, 0)),
            pl.BlockSpec((1, d, HBLK), lambda e, hb, cnt: (e, 0, hb)),
            pl.BlockSpec((1, 1, 1, HBLK), lambda e, hb, cnt: (e, hb, 0, 0)),
            pl.BlockSpec((1, HBLK, d), lambda e, hb, cnt: (e, hb, 0)),
            pl.BlockSpec((1, 1, d), lambda e, hb, cnt: (e, 0, 0)),
        ],
        out_specs=pl.BlockSpec((cap, d), lambda e, hb, cnt: (e, 0)),
        scratch_shapes=[
            pltpu.VMEM((cap, d), jnp.bfloat16),
        ],
    )
    out_buf = pl.pallas_call(
        ffn_body,
        grid_spec=grid_spec,
        out_shape=jax.ShapeDtypeStruct((e_num * cap, d), f32),
    )(cnt_tot, buf, W1, b1.reshape(e_num, hb_num, 1, HBLK),
      W2, b2.reshape(e_num, 1, d))

    r = _gather_sc(nchunk, t_num, d, e_num * cap)(out_buf,
                                                  dstc.reshape(2, t_num))
    r = r.reshape(2, t_num, d)
    wt = w.reshape(2, t_num, 1)

    y = pl.pallas_call(
        _combine_body,
        grid=(t_num // BT,),
        in_specs=[
            pl.BlockSpec((2, BT, 1), lambda i: (0, i, 0)),
            pl.BlockSpec((2, BT, d), lambda i: (0, i, 0)),
        ],
        out_specs=pl.BlockSpec((BT, d), lambda i: (i, 0)),
        out_shape=jax.ShapeDtypeStruct((t_num, d), f32),
    )(wt, r)

    return y, laux[0, 0]


# speedup vs baseline: 1.9239x; 1.0793x over previous
import functools

import jax
import jax.numpy as jnp
from jax import lax
from jax.experimental import pallas as pl
from jax.experimental.pallas import tpu as pltpu
from jax.experimental.pallas import tpu_sc as plsc

CAP_FACTOR_NUM = 5
CAP_FACTOR_DEN = 4

NC = 2
NS = 16
NW = NC * NS

PB = 1024
HBLK = 1024
BT = 512
C = 32


def _gate_body(x_ref, wg_ref, i1_ref, i2_ref, v1_ref, v2_ref, gs_ref):
    xb = x_ref[...].astype(jnp.bfloat16)
    wb = wg_ref[...].astype(jnp.bfloat16)
    logits = jnp.dot(xb, wb, preferred_element_type=jnp.float32)
    m = jnp.max(logits, axis=1, keepdims=True)
    p = jnp.exp(logits - m)
    g = p / jnp.sum(p, axis=1, keepdims=True)
    e = g.shape[1]
    cols = lax.broadcasted_iota(jnp.int32, g.shape, 1)
    big = jnp.int32(e)
    v1 = jnp.max(g, axis=1, keepdims=True)
    i1 = jnp.min(jnp.where(g == v1, cols, big), axis=1, keepdims=True)
    g2 = jnp.where(cols == i1, -jnp.inf, g)
    v2 = jnp.max(g2, axis=1, keepdims=True)
    i2 = jnp.min(jnp.where(g2 == v2, cols, big), axis=1, keepdims=True)
    i1_ref[...] = i1
    i2_ref[...] = i2
    v1_ref[...] = v1
    v2_ref[...] = v2
    gs_ref[...] = jnp.sum(g, axis=0, keepdims=True)[None]


def _positions_body(i1_ref, i2_ref, v1_ref, v2_ref, gs_ref,
                    dstd_ref, dstc_ref, w_ref, cnt_ref, laux_ref,
                    tri_ref, off_ref, cnt1_ref, *, cap, e_num, t_num, nb):
    k = pl.program_id(0)
    b = pl.program_id(1)

    @pl.when(jnp.logical_and(k == 0, b == 0))
    def _init():
        rr = lax.broadcasted_iota(jnp.int32, (PB, PB), 0)
        cc = lax.broadcasted_iota(jnp.int32, (PB, PB), 1)
        tri_ref[...] = (rr >= cc).astype(jnp.bfloat16)
        off_ref[...] = jnp.zeros_like(off_ref)
        cnt1_ref[...] = jnp.zeros_like(cnt1_ref)

    ik = jnp.where(k == 0, i1_ref[...], i2_ref[...])
    vk = jnp.where(k == 0, v1_ref[...], v2_ref[...])
    cols = lax.broadcasted_iota(jnp.int32, (PB, e_num), 1)
    mh = (cols == ik)
    m_bf = mh.astype(jnp.bfloat16)
    m_f = mh.astype(jnp.float32)
    cs = jnp.dot(tri_ref[...], m_bf, preferred_element_type=jnp.float32)
    off = off_ref[...]
    pos_in = cs - 1.0 + off
    posf = jnp.sum(m_f * pos_in, axis=1, keepdims=True)
    posi = posf.astype(jnp.int32)
    valid = posi < cap
    dummy = jnp.int32(e_num * cap)
    dstd_ref[...] = jnp.where(valid, ik * cap + posi, dummy)[None, None]
    dstc_ref[...] = (ik * cap + jnp.minimum(posi, cap - 1))[None, None]
    wtk = vk * valid.astype(jnp.float32)
    w_ref[...] = jnp.broadcast_to(wtk, (wtk.shape[0], 16))[None, None]
    off = off + cs[PB - 1:PB, :]
    off_ref[...] = off
    cnt_ref[...] = off[None]

    @pl.when(jnp.logical_and(k == 0, b == nb - 1))
    def _snap():
        cnt1_ref[...] = off

    gs_tot = jnp.sum(gs_ref[...][:, 0, :], axis=0, keepdims=True)
    laux = jnp.sum(gs_tot * cnt1_ref[...], axis=1, keepdims=True)
    laux_ref[...] = laux * (e_num / (float(t_num) * float(t_num)))


def _ffn_body(cnt_ref, buf_ref, w1_ref, b1_ref, w2_ref, b2_ref, out_ref,
              xb_ref, *, cap, d, hb_num):
    e = pl.program_id(0)
    hb = pl.program_id(1)

    @pl.when(hb == 0)
    def _load():
        n = jnp.minimum(cnt_ref[e], cap)
        rows = lax.broadcasted_iota(jnp.int32, (cap, d), 0)
        xb_ref[...] = jnp.where(rows < n, buf_ref[...],
                                0.0).astype(jnp.bfloat16)

    w1b = w1_ref[0].astype(jnp.bfloat16)
    w2b = w2_ref[0].astype(jnp.bfloat16)
    b1v = b1_ref[0, 0]
    half = cap // 2
    h0 = jnp.dot(xb_ref[pl.ds(0, half), :], w1b,
                 preferred_element_type=jnp.float32)
    g0 = jax.nn.gelu(h0 + b1v, approximate=True).astype(jnp.bfloat16)
    h1 = jnp.dot(xb_ref[pl.ds(half, half), :], w1b,
                 preferred_element_type=jnp.float32)
    p0 = jnp.dot(g0, w2b, preferred_element_type=jnp.float32)
    g1 = jax.nn.gelu(h1 + b1v, approximate=True).astype(jnp.bfloat16)
    p1 = jnp.dot(g1, w2b, preferred_element_type=jnp.float32)

    @pl.when(hb == 0)
    def _first():
        b2v = b2_ref[0]
        out_ref[pl.ds(0, half), :] = p0 + b2v
        out_ref[pl.ds(half, half), :] = p1 + b2v

    @pl.when(hb > 0)
    def _rest():
        out_ref[pl.ds(0, half), :] += p0
        out_ref[pl.ds(half, half), :] += p1


def _combine_body(w_ref, r_ref, y_ref):
    y_ref[...] = (w_ref[0] * r_ref[0] + w_ref[1] * r_ref[1])


def _dispatch_sc(nchunk, t_num, d, buf_rows):
    mesh = plsc.VectorSubcoreMesh(core_axis_name="c", subcore_axis_name="s",
                                  num_cores=NC, num_subcores=NS)
    tpw = t_num // NW

    @functools.partial(
        pl.kernel,
        out_type=jax.ShapeDtypeStruct((buf_rows, d), jnp.float32),
        mesh=mesh,
        scratch_types=[
            pltpu.VMEM((2, C, d), jnp.float32),
            pltpu.VMEM((2 * nchunk, C), jnp.int32),
            pltpu.SemaphoreType.DMA,
            pltpu.SemaphoreType.DMA,
            pltpu.SemaphoreType.DMA,
        ],
    )
    def dispatch(x_hbm, idx_hbm, buf_hbm, xbuf, idx_v, isem, lsem, ssem):
        wid = lax.axis_index("s") * NC + lax.axis_index("c")
        icps = []
        for c in range(nchunk):
            for k in range(2):
                icps.append(pltpu.async_copy(
                    idx_hbm.at[k, pl.ds(wid * tpw + c * C, C)],
                    idx_v.at[2 * c + k], isem))
        for cp in icps:
            cp.wait()
        ld = pltpu.async_copy(x_hbm.at[pl.ds(wid * tpw, C)],
                              xbuf.at[0], lsem)
        for c in range(nchunk):
            ld.wait()
            a0 = pltpu.async_copy(xbuf.at[c % 2],
                                  buf_hbm.at[idx_v.at[2 * c]], ssem)
            a1 = pltpu.async_copy(xbuf.at[c % 2],
                                  buf_hbm.at[idx_v.at[2 * c + 1]], ssem)
            if c + 1 < nchunk:
                ld = pltpu.async_copy(
                    x_hbm.at[pl.ds(wid * tpw + (c + 1) * C, C)],
                    xbuf.at[(c + 1) % 2], lsem)
            a0.wait()
            a1.wait()

    return dispatch


def _combine_sc(nchunk, t_num, d, out_rows):
    mesh = plsc.VectorSubcoreMesh(core_axis_name="c", subcore_axis_name="s",
                                  num_cores=NC, num_subcores=NS)
    tpw = t_num // NW
    cc = 8
    ncc = tpw // cc
    nseg = d // 16

    @functools.partial(
        pl.kernel,
        out_type=jax.ShapeDtypeStruct((t_num, d), jnp.float32),
        mesh=mesh,
        scratch_types=[
            pltpu.VMEM((2, cc, d), jnp.float32),
            pltpu.VMEM((2, cc, d), jnp.float32),
            pltpu.VMEM((2 * ncc, cc), jnp.int32),
            pltpu.VMEM((2, tpw, 16), jnp.float32),
            pltpu.SemaphoreType.DMA,
            pltpu.SemaphoreType.DMA,
        ],
    )
    def combine(ob_hbm, idx_hbm, w_hbm, y_hbm, r0, r1, idx_v, wv,
                isem, gsem):
        wid = lax.axis_index("s") * NC + lax.axis_index("c")
        base = wid * tpw
        icps = []
        for c in range(ncc):
            for k in range(2):
                icps.append(pltpu.async_copy(
                    idx_hbm.at[k, pl.ds(base + c * cc, cc)],
                    idx_v.at[2 * c + k], isem))
        for k in range(2):
            icps.append(pltpu.async_copy(w_hbm.at[k, pl.ds(base, tpw)],
                                         wv.at[k], isem))
        for cp in icps:
            cp.wait()
        g0 = pltpu.async_copy(ob_hbm.at[idx_v.at[0]], r0.at[0], gsem)
        g1 = pltpu.async_copy(ob_hbm.at[idx_v.at[1]], r1.at[0], gsem)
        for c in range(ncc):
            g0.wait()
            g1.wait()
            if c + 1 < ncc:
                g0 = pltpu.async_copy(ob_hbm.at[idx_v.at[2 * c + 2]],
                                      r0.at[(c + 1) % 2], gsem)
                g1 = pltpu.async_copy(ob_hbm.at[idx_v.at[2 * c + 3]],
                                      r1.at[(c + 1) % 2], gsem)
            cb = c % 2

            def tok_body(t, _):
                ct = c * cc + t
                w0 = wv[0, ct, :]
                w1 = wv[1, ct, :]

                @plsc.parallel_loop(0, nseg, unroll=8)
                def _seg(j):
                    sl = pl.ds(j * 16, 16)
                    a = r0[cb, t, sl]
                    b = r1[cb, t, sl]
                    r0[cb, t, sl] = a * w0 + b * w1

                return 0

            lax.fori_loop(0, cc, tok_body, 0)
            pltpu.sync_copy(r0.at[cb], y_hbm.at[pl.ds(base + c * cc, cc)])

    return combine


def kernel(x, Wg, W1, b1, W2, b2):
    t_num, d = x.shape
    e_num = Wg.shape[1]
    h_num = W1.shape[2]
    cap = 2 * ((CAP_FACTOR_NUM * ((t_num + e_num - 1) // e_num))
               // CAP_FACTOR_DEN)
    nb = t_num // PB
    hb_num = h_num // HBLK
    nchunk = t_num // (NW * C)
    buf_rows = e_num * cap + 8

    f32 = jnp.float32
    i32 = jnp.int32

    i1, i2, v1, v2, gs = pl.pallas_call(
        _gate_body,
        grid=(nb,),
        in_specs=[
            pl.BlockSpec((PB, d), lambda b: (b, 0)),
            pl.BlockSpec((d, e_num), lambda b: (0, 0)),
        ],
        out_specs=[
            pl.BlockSpec((PB, 1), lambda b: (b, 0)),
            pl.BlockSpec((PB, 1), lambda b: (b, 0)),
            pl.BlockSpec((PB, 1), lambda b: (b, 0)),
            pl.BlockSpec((PB, 1), lambda b: (b, 0)),
            pl.BlockSpec((1, 1, e_num), lambda b: (b, 0, 0)),
        ],
        out_shape=[
            jax.ShapeDtypeStruct((t_num, 1), i32),
            jax.ShapeDtypeStruct((t_num, 1), i32),
            jax.ShapeDtypeStruct((t_num, 1), f32),
            jax.ShapeDtypeStruct((t_num, 1), f32),
            jax.ShapeDtypeStruct((nb, 1, e_num), f32),
        ],
    )(x, Wg)

    pos_body = functools.partial(_positions_body, cap=cap, e_num=e_num,
                                 t_num=t_num, nb=nb)
    dstd, dstc, w, cnt, laux = pl.pallas_call(
        pos_body,
        grid=(2, nb),
        in_specs=[
            pl.BlockSpec((PB, 1), lambda k, b: (b, 0)),
            pl.BlockSpec((PB, 1), lambda k, b: (b, 0)),
            pl.BlockSpec((PB, 1), lambda k, b: (b, 0)),
            pl.BlockSpec((PB, 1), lambda k, b: (b, 0)),
            pl.BlockSpec((nb, 1, e_num), lambda k, b: (0, 0, 0)),
        ],
        out_specs=[
            pl.BlockSpec((1, 1, PB, 1), lambda k, b: (k, b, 0, 0)),
            pl.BlockSpec((1, 1, PB, 1), lambda k, b: (k, b, 0, 0)),
            pl.BlockSpec((1, 1, PB, 16), lambda k, b: (k, b, 0, 0)),
            pl.BlockSpec((1, 1, e_num), lambda k, b: (k, 0, 0)),
            pl.BlockSpec((1, 1), lambda k, b: (0, 0)),
        ],
        out_shape=[
            jax.ShapeDtypeStruct((2, nb, PB, 1), i32),
            jax.ShapeDtypeStruct((2, nb, PB, 1), i32),
            jax.ShapeDtypeStruct((2, nb, PB, 16), f32),
            jax.ShapeDtypeStruct((2, 1, e_num), f32),
            jax.ShapeDtypeStruct((1, 1), f32),
        ],
        scratch_shapes=[
            pltpu.VMEM((PB, PB), jnp.bfloat16),
            pltpu.VMEM((1, e_num), f32),
            pltpu.VMEM((1, e_num), f32),
        ],
    )(i1, i2, v1, v2, gs)

    cnt_tot = cnt[1, 0].astype(i32)

    buf = _dispatch_sc(nchunk, t_num, d, buf_rows)(x, dstd.reshape(2, t_num))

    ffn_body = functools.partial(_ffn_body, cap=cap, d=d, hb_num=hb_num)
    grid_spec = pltpu.PrefetchScalarGridSpec(
        num_scalar_prefetch=1,
        grid=(e_num, hb_num),
        in_specs=[
            pl.BlockSpec((cap, d), lambda e, hb, cnt: (e, 0)),
            pl.BlockSpec((1, d, HBLK), lambda e, hb, cnt: (e, 0, hb)),
            pl.BlockSpec((1, 1, 1, HBLK), lambda e, hb, cnt: (e, hb, 0, 0)),
            pl.BlockSpec((1, HBLK, d), lambda e, hb, cnt: (e, hb, 0)),
            pl.BlockSpec((1, 1, d), lambda e, hb, cnt: (e, 0, 0)),
        ],
        out_specs=pl.BlockSpec((cap, d), lambda e, hb, cnt: (e, 0)),
        scratch_shapes=[
            pltpu.VMEM((cap, d), jnp.bfloat16),
        ],
    )
    out_buf = pl.pallas_call(
        ffn_body,
        grid_spec=grid_spec,
        out_shape=jax.ShapeDtypeStruct((e_num * cap, d), f32),
    )(cnt_tot, buf, W1, b1.reshape(e_num, hb_num, 1, HBLK),
      W2, b2.reshape(e_num, 1, d))

    y = _combine_sc(nchunk, t_num, d, e_num * cap)(
        out_buf, dstc.reshape(2, t_num), w.reshape(2, t_num, 16))

    return y, laux[0, 0]
